# baseline (speedup 1.0000x reference)
import functools

import jax
import jax.numpy as jnp
from jax import lax
from jax.experimental import pallas as pl
from jax.experimental.pallas import tpu as pltpu
from jax.experimental.pallas import tpu_sc as plsc

_NUM_POS = 8192
_EMBED_DIM = 768
_B = 4
_S = 8192
_TOTAL = _B * _S

_NC = 2
_NS = 16
_NW = _NC * _NS
_ROWS_PER_W = _TOTAL // _NW
_CHUNK = 32
_N_CHUNKS = _ROWS_PER_W // _CHUNK
_NBUF = 2
_N_GROUPS = _N_CHUNKS // _NBUF

_WROWS_PER_W = _NUM_POS // _NW
_WCHUNK = 32
_N_WCHUNKS = _WROWS_PER_W // _WCHUNK

_mesh = plsc.VectorSubcoreMesh(core_axis_name="c", subcore_axis_name="s")


@functools.partial(
    pl.kernel,
    mesh=_mesh,
    out_type=(
        jax.ShapeDtypeStruct((_TOTAL, _EMBED_DIM), jnp.float32),
        jax.ShapeDtypeStruct((_NUM_POS, _EMBED_DIM), jnp.float32),
    ),
    scratch_types=[
        pltpu.VMEM((_N_CHUNKS, _CHUNK), jnp.int32),
        pltpu.VMEM((_NBUF, _CHUNK, _EMBED_DIM), jnp.float32),
        pltpu.VMEM((2, _WCHUNK, _EMBED_DIM), jnp.float32),
        pltpu.SemaphoreType.DMA,
        pltpu.SemaphoreType.DMA,
        pltpu.SemaphoreType.DMA,
        pltpu.SemaphoreType.DMA,
    ],
)
def _gather_rows(idx_hbm, table_hbm, out_hbm, wout_hbm,
                 idx_v, rows_v, wbuf, sg, sw, swi, swo):
    sid = lax.axis_index("s")
    wid = sid * _NC + lax.axis_index("c")
    base = wid * _ROWS_PER_W
    wbase = wid * _WROWS_PER_W

    pltpu.sync_copy(idx_hbm.at[wid], idx_v)

    def _wc_in(k):
        pltpu.async_copy(table_hbm.at[pl.ds(wbase + k * _WCHUNK, _WCHUNK)],
                         wbuf.at[k % 2], swi)

    def _wc_wait_in():
        pltpu.make_async_copy(table_hbm.at[pl.ds(0, _WCHUNK)],
                              wbuf.at[0], swi).wait()

    def _wc_out(k):
        pltpu.async_copy(wbuf.at[k % 2],
                         wout_hbm.at[pl.ds(wbase + k * _WCHUNK, _WCHUNK)],
                         swo)

    def _wc_wait_out():
        pltpu.make_async_copy(wbuf.at[0],
                              wout_hbm.at[pl.ds(0, _WCHUNK)], swo).wait()

    for b in range(_NBUF):
        pltpu.async_copy(table_hbm.at[idx_v.at[b]], rows_v.at[b], sg)

    def _wait_gather(b):
        pltpu.make_async_copy(
            table_hbm.at[idx_v.at[0]], rows_v.at[b], sg).wait()

    def _write_out(i, b):
        dst = out_hbm.at[pl.ds(base + i * _CHUNK, _CHUNK)]
        pltpu.async_copy(rows_v.at[b], dst, sw)
        return dst

    def _group(g, carry):
        for b in range(_NBUF):
            i = g * _NBUF + b
            _wait_gather(b)
            dst = _write_out(i, b)
            pltpu.make_async_copy(rows_v.at[b], dst, sw).wait()
            pltpu.async_copy(
                table_hbm.at[idx_v.at[i + _NBUF]], rows_v.at[b], sg)
        @pl.when(g % 2 == 1)
        def _advance():
            k = g // 2
            _wc_wait_in()
            _wc_out(k)

            @pl.when(k >= 1)
            def _drain_prev():
                _wc_wait_out()

            _wc_in(k + 1)
        return carry

    _wc_in(0)
    lax.fori_loop(0, _N_GROUPS - 1, _group, 0)

    dsts = []
    for b in range(_NBUF):
        i = (_N_GROUPS - 1) * _NBUF + b
        _wait_gather(b)
        dsts.append((b, _write_out(i, b)))

    _wc_wait_in()
    _wc_out(_N_WCHUNKS - 1)
    _wc_wait_out()
    _wc_wait_out()
    for b, dst in dsts:
        pltpu.make_async_copy(rows_v.at[b], dst, sw).wait()


def kernel(inputs, pos_embed_weights):
    idx = inputs.astype(jnp.int32).reshape(_NW, _N_CHUNKS, _CHUNK)
    out, w_out = _gather_rows(idx, pos_embed_weights)
    return out.reshape(_B, _S, _EMBED_DIM), w_out


# speedup vs baseline: 1.0123x; 1.0123x over previous
import functools

import jax
import jax.numpy as jnp
from jax import lax
from jax.experimental import pallas as pl
from jax.experimental.pallas import tpu as pltpu
from jax.experimental.pallas import tpu_sc as plsc

_NUM_POS = 8192
_EMBED_DIM = 768
_B = 4
_S = 8192
_TOTAL = _B * _S

_NC = 2
_NS = 16
_NW = _NC * _NS
_ROWS_PER_W = _TOTAL // _NW
_CHUNK = 32
_N_CHUNKS = _ROWS_PER_W // _CHUNK
_NBUF = 4
_N_GROUPS = _N_CHUNKS // _NBUF

_mesh = plsc.VectorSubcoreMesh(core_axis_name="c", subcore_axis_name="s")


@functools.partial(
    pl.kernel,
    mesh=_mesh,
    out_type=jax.ShapeDtypeStruct((_TOTAL, _EMBED_DIM), jnp.float32),
    scratch_types=[
        pltpu.VMEM((_N_CHUNKS, _CHUNK), jnp.int32),
        pltpu.VMEM((_NBUF, _CHUNK, _EMBED_DIM), jnp.float32),
        pltpu.SemaphoreType.DMA,
        pltpu.SemaphoreType.DMA,
    ],
)
def _gather_rows(idx_hbm, table_hbm, out_hbm, idx_v, rows_v, sg, sw):
    sid = lax.axis_index("s")
    wid = sid * _NC + lax.axis_index("c")
    base = wid * _ROWS_PER_W

    pltpu.sync_copy(idx_hbm.at[wid], idx_v)

    for b in range(_NBUF):
        pltpu.async_copy(table_hbm.at[idx_v.at[b]], rows_v.at[b], sg)

    def _wait_gather(b):
        pltpu.make_async_copy(
            table_hbm.at[idx_v.at[0]], rows_v.at[b], sg).wait()

    def _write_out(i, b):
        dst = out_hbm.at[pl.ds(base + i * _CHUNK, _CHUNK)]
        pltpu.async_copy(rows_v.at[b], dst, sw)
        return dst

    def _group(g, carry):
        for b in range(_NBUF):
            i = g * _NBUF + b
            _wait_gather(b)
            dst = _write_out(i, b)
            pltpu.make_async_copy(rows_v.at[b], dst, sw).wait()
            pltpu.async_copy(
                table_hbm.at[idx_v.at[i + _NBUF]], rows_v.at[b], sg)
        return carry

    lax.fori_loop(0, _N_GROUPS - 1, _group, 0)

    dsts = []
    for b in range(_NBUF):
        i = (_N_GROUPS - 1) * _NBUF + b
        _wait_gather(b)
        dsts.append((b, _write_out(i, b)))
    for b, dst in dsts:
        pltpu.make_async_copy(rows_v.at[b], dst, sw).wait()


def _alias_body(w_ref, o_ref):
    pass


def _weights_passthrough(w):
    return pl.pallas_call(
        _alias_body,
        in_specs=[pl.BlockSpec(memory_space=pltpu.HBM)],
        out_specs=pl.BlockSpec(memory_space=pltpu.HBM),
        out_shape=jax.ShapeDtypeStruct((_NUM_POS, _EMBED_DIM), jnp.float32),
        input_output_aliases={0: 0},
    )(w)


def kernel(inputs, pos_embed_weights):
    idx = inputs.astype(jnp.int32).reshape(_NW, _N_CHUNKS, _CHUNK)
    out = _gather_rows(idx, pos_embed_weights)
    w_out = _weights_passthrough(pos_embed_weights)
    return out.reshape(_B, _S, _EMBED_DIM), w_out


# speedup vs baseline: 1.0359x; 1.0233x over previous
import functools

import jax
import jax.numpy as jnp
from jax import lax
from jax.experimental import pallas as pl
from jax.experimental.pallas import tpu as pltpu
from jax.experimental.pallas import tpu_sc as plsc

_NUM_POS = 8192
_EMBED_DIM = 768
_B = 4
_S = 8192
_TOTAL = _B * _S

_NC = 2
_NS = 16
_NW = _NC * _NS
_ROWS_PER_W = _TOTAL // _NW
_CHUNK = 32
_N_CHUNKS = _ROWS_PER_W // _CHUNK
_NBUF = 4

_mesh = plsc.VectorSubcoreMesh(core_axis_name="c", subcore_axis_name="s")


@functools.partial(
    pl.kernel,
    mesh=_mesh,
    out_type=jax.ShapeDtypeStruct((_TOTAL, _EMBED_DIM), jnp.float32),
    scratch_types=[
        pltpu.VMEM((_N_CHUNKS, _CHUNK), jnp.int32),
        pltpu.VMEM((_NBUF, _CHUNK, _EMBED_DIM), jnp.float32),
        pltpu.SemaphoreType.DMA,
        pltpu.SemaphoreType.DMA,
    ],
)
def _gather_rows(idx_hbm, table_hbm, out_hbm, idx_v, rows_v, sg, sw):
    sid = lax.axis_index("s")
    wid = sid * _NC + lax.axis_index("c")
    base = wid * _ROWS_PER_W

    pltpu.sync_copy(idx_hbm.at[wid], idx_v)

    def _gather(i, b):
        pltpu.async_copy(table_hbm.at[idx_v.at[i]], rows_v.at[b], sg)

    def _wait_gather():
        pltpu.make_async_copy(
            table_hbm.at[idx_v.at[0]], rows_v.at[0], sg).wait()

    def _write_out(i, b):
        pltpu.async_copy(rows_v.at[b],
                         out_hbm.at[pl.ds(base + i * _CHUNK, _CHUNK)], sw)

    def _wait_write():
        pltpu.make_async_copy(
            rows_v.at[0], out_hbm.at[pl.ds(base, _CHUNK)], sw).wait()

    for b in range(_NBUF):
        _gather(b, b)

    def _body(i, carry):
        b = lax.rem(i, _NBUF)
        _wait_gather()
        _write_out(i, b)
        _wait_write()
        _gather(i + _NBUF, b)
        return carry

    lax.fori_loop(0, _N_CHUNKS - _NBUF, _body, 0)

    def _tail(i, carry):
        _wait_gather()
        _write_out(i, lax.rem(i, _NBUF))
        _wait_write()
        return carry

    lax.fori_loop(_N_CHUNKS - _NBUF, _N_CHUNKS, _tail, 0)


def _copy_body(w_ref, o_ref):
    o_ref[...] = w_ref[...]


def _weights_passthrough(w):
    return pl.pallas_call(
        _copy_body,
        grid=(16,),
        in_specs=[pl.BlockSpec((_NUM_POS // 16, _EMBED_DIM),
                               lambda i: (i, 0))],
        out_specs=pl.BlockSpec((_NUM_POS // 16, _EMBED_DIM),
                               lambda i: (i, 0)),
        out_shape=jax.ShapeDtypeStruct((_NUM_POS, _EMBED_DIM), jnp.float32),
    )(w)


def kernel(inputs, pos_embed_weights):
    idx = inputs.astype(jnp.int32).reshape(_NW, _N_CHUNKS, _CHUNK)
    out = _gather_rows(idx, pos_embed_weights)
    w_out = _weights_passthrough(pos_embed_weights)
    return out.reshape(_B, _S, _EMBED_DIM), w_out
